# baseline (speedup 1.0000x reference)
import functools

import jax
import jax.numpy as jnp
from jax import lax
from jax.experimental import pallas as pl
from jax.experimental.pallas import tpu as pltpu
from jax.experimental.pallas import tpu_sc as plsc

N_IN_ROWS = 50000
N_OUT_ROWS = 400000
FEAT = 128

NUM_CORES = 2
NUM_SUBCORES = 16
NUM_WORKERS = NUM_CORES * NUM_SUBCORES

CHUNK = 128
NUM_CHUNKS = N_OUT_ROWS // CHUNK
STEPS = 98
ROWS_PER_WORKER = STEPS * CHUNK
MAIN_ROWS = 87 * CHUNK
WRAP_ROWS = ROWS_PER_WORKER - MAIN_ROWS
NBUF = 7
DEPTH = 2


def _gather_body(table_hbm, idx_hbm, out_hbm, idx_v, rowbufs, isem, gsems, wsems):
    wid = lax.axis_index("s") * NUM_CORES + lax.axis_index("c")
    base = wid * ROWS_PER_WORKER

    def chunk_row0(k):
        c = wid * STEPS + k
        c = jnp.where(c < NUM_CHUNKS, c, c - NUM_CHUNKS)
        return c * CHUNK

    wrap_base = jnp.where(base + ROWS_PER_WORKER <= N_OUT_ROWS, base + MAIN_ROWS, 0)

    def main_idx_copy():
        return pltpu.make_async_copy(
            idx_hbm.at[pl.ds(base, MAIN_ROWS)], idx_v.at[pl.ds(0, MAIN_ROWS)], isem
        )

    def wrap_idx_copy():
        return pltpu.make_async_copy(
            idx_hbm.at[pl.ds(wrap_base, WRAP_ROWS)],
            idx_v.at[pl.ds(MAIN_ROWS, WRAP_ROWS)],
            isem,
        )

    main_idx_copy().start()
    wrap_idx_copy().start()
    main_idx_copy().wait()
    wrap_idx_copy().wait()

    def gather_copy(k, b):
        return pltpu.make_async_copy(
            table_hbm.at[idx_v.at[pl.ds(k * CHUNK, CHUNK)]], rowbufs[b], gsems[b]
        )

    def wb_copy(k, b):
        return pltpu.make_async_copy(
            rowbufs[b], out_hbm.at[pl.ds(chunk_row0(k), CHUNK)], wsems[b]
        )

    def step(k_static_b, g):
        b = k_static_b
        k = g * NBUF + b

        @pl.when(g >= 1)
        def _wait_buf_free():
            wb_copy(0, b).wait()

        gather_copy(k, b).start()

        b2 = (b - DEPTH) % NBUF
        p = k - DEPTH

        def _writeback():
            gather_copy(0, b2).wait()
            wb_copy(p, b2).start()

        if b >= DEPTH:
            _writeback()
        else:
            pl.when(g >= 1)(_writeback)

    def ring_pass(g, _):
        for b in range(NBUF):
            step(b, g)
        return 0

    lax.fori_loop(0, STEPS // NBUF, ring_pass, 0)

    for p in range(STEPS - DEPTH, STEPS):
        pb = p % NBUF
        gather_copy(0, pb).wait()
        wb_copy(p, pb).start()
    for b in range(NBUF):
        wb_copy(0, b).wait()


@jax.jit
def _unpool_gather(table, idx):
    mesh = plsc.VectorSubcoreMesh(core_axis_name="c", subcore_axis_name="s")
    run = functools.partial(
        pl.kernel,
        mesh=mesh,
        out_type=jax.ShapeDtypeStruct((N_OUT_ROWS, FEAT), jnp.float32),
        scratch_types=[
            pltpu.VMEM((ROWS_PER_WORKER,), jnp.int32),
            [pltpu.VMEM((CHUNK, FEAT), jnp.float32) for _ in range(NBUF)],
            pltpu.SemaphoreType.DMA,
            [pltpu.SemaphoreType.DMA for _ in range(NBUF)],
            [pltpu.SemaphoreType.DMA for _ in range(NBUF)],
        ],
    )(_gather_body)
    return run(table, idx)


def kernel(input_features, unpool_map):
    return _unpool_gather(input_features, unpool_map.astype(jnp.int32))
